# baseline (speedup 1.0000x reference)
import jax
import jax.numpy as jnp
from jax.experimental import pallas as pl
from jax.experimental.pallas import tpu as pltpu

_BC = 8192


def _mlp_t(obs_ref, act_ref, w1a_ref, w1b_ref, b1_ref, w2_ref, b2_ref,
           w3_ref, b3_ref, w4_ref, b4_ref, out_ref):
    h = w1a_ref[...] @ obs_ref[...] + w1b_ref[...] @ act_ref[...]
    h = jnp.maximum(h + b1_ref[...], 0.0)
    h = jnp.maximum(w2_ref[...] @ h + b2_ref[...], 0.0)
    h = jnp.maximum(w3_ref[...] @ h + b3_ref[...], 0.0)
    out_ref[...] = w4_ref[...] @ h + b4_ref[...]


@jax.jit
def kernel(obs, actions, W1, b1, W2, b2, W3, b3, W4, b4):
    B, n_obs = obs.shape
    n_act = actions.shape[1]
    num_atoms = W4.shape[1]

    def full(shape):
        return pl.BlockSpec(shape, lambda i: (0, 0))

    return pl.pallas_call(
        _mlp_t,
        grid=(B // _BC,),
        in_specs=[
            pl.BlockSpec((n_obs, _BC), lambda i: (0, i)),
            pl.BlockSpec((n_act, _BC), lambda i: (0, i)),
            full((128, n_obs)),
            full((128, n_act)),
            full((128, 1)),
            full((64, 128)),
            full((64, 1)),
            full((32, 64)),
            full((32, 1)),
            full((num_atoms, 32)),
            full((num_atoms, 1)),
        ],
        out_specs=pl.BlockSpec((num_atoms, _BC), lambda i: (0, i)),
        out_shape=jax.ShapeDtypeStruct((num_atoms, B), jnp.float32),
        compiler_params=pltpu.CompilerParams(
            dimension_semantics=("parallel",)),
    )(obs.T, actions.T,
      W1[:n_obs].T, W1[n_obs:].T, b1[:, None],
      W2.T, b2[:, None], W3.T, b3[:, None], W4.T, b4[:, None]).T


# speedup vs baseline: 1.0717x; 1.0717x over previous
import jax
import jax.numpy as jnp
from jax.experimental import pallas as pl
from jax.experimental.pallas import tpu as pltpu

_BC = 16384


def _mlp_t(obs_ref, act_ref, w1a_ref, w1b_ref, b1_ref, w2_ref, b2_ref,
           w3_ref, b3_ref, w4_ref, b4_ref, out_ref):
    h = w1a_ref[...] @ obs_ref[...] + w1b_ref[...] @ act_ref[...]
    h = jnp.maximum(h + b1_ref[...], 0.0)
    h = jnp.maximum(w2_ref[...] @ h + b2_ref[...], 0.0)
    h = jnp.maximum(w3_ref[...] @ h + b3_ref[...], 0.0)
    out_ref[...] = w4_ref[...] @ h + b4_ref[...]


@jax.jit
def kernel(obs, actions, W1, b1, W2, b2, W3, b3, W4, b4):
    B, n_obs = obs.shape
    n_act = actions.shape[1]
    num_atoms = W4.shape[1]

    def full(shape):
        return pl.BlockSpec(shape, lambda i: (0, 0))

    return pl.pallas_call(
        _mlp_t,
        grid=(B // _BC,),
        in_specs=[
            pl.BlockSpec((n_obs, _BC), lambda i: (0, i)),
            pl.BlockSpec((n_act, _BC), lambda i: (0, i)),
            full((128, n_obs)),
            full((128, n_act)),
            full((128, 1)),
            full((64, 128)),
            full((64, 1)),
            full((32, 64)),
            full((32, 1)),
            full((num_atoms, 32)),
            full((num_atoms, 1)),
        ],
        out_specs=pl.BlockSpec((num_atoms, _BC), lambda i: (0, i)),
        out_shape=jax.ShapeDtypeStruct((num_atoms, B), jnp.float32),
        compiler_params=pltpu.CompilerParams(
            dimension_semantics=("parallel",)),
    )(obs.T, actions.T,
      W1[:n_obs].T, W1[n_obs:].T, b1[:, None],
      W2.T, b2[:, None], W3.T, b3[:, None], W4.T, b4[:, None]).T
